# Initial kernel scaffold; baseline (speedup 1.0000x reference)
import jax
import jax.numpy as jnp
from jax.experimental import pallas as pl


def kernel(x, table):
    raise NotImplementedError("write your pallas kernel here")


# speedup vs baseline: 2.9649x; 2.9649x over previous
import functools

import jax
import jax.numpy as jnp
from jax import lax
from jax.experimental import pallas as pl
from jax.experimental.pallas import tpu as pltpu
from jax.experimental.pallas import tpu_sc as plsc

_EMBED_DIM = 128
_CHUNK = 128


@functools.lru_cache(maxsize=None)
def _make_gather(batch_total, vocab, embed_dim):
    info = plsc.get_sparse_core_info()
    num_cores = info.num_cores
    num_workers = info.num_cores * info.num_subcores
    b_per_w = batch_total // num_workers
    nchunk = b_per_w // _CHUNK
    assert b_per_w % _CHUNK == 0

    mesh = plsc.VectorSubcoreMesh(core_axis_name="c", subcore_axis_name="s")

    @functools.partial(
        pl.kernel,
        mesh=mesh,
        out_type=jax.ShapeDtypeStruct((batch_total, embed_dim), jnp.float32),
        scratch_types=[
            pltpu.VMEM((nchunk, _CHUNK), jnp.int32),
            pltpu.VMEM((_CHUNK, embed_dim), jnp.float32),
            pltpu.SemaphoreType.DMA,
        ],
    )
    def gather_kernel(idx_hbm, table_hbm, out_hbm, idx_v, buf, sem):
        wid = lax.axis_index("s") * num_cores + lax.axis_index("c")
        base = wid * b_per_w
        pltpu.sync_copy(idx_hbm.at[wid], idx_v)

        def body(g, carry):
            pltpu.async_copy(table_hbm.at[idx_v.at[g]], buf, sem).wait()
            pltpu.sync_copy(buf, out_hbm.at[pl.ds(base + g * _CHUNK, _CHUNK)])
            return carry

        lax.fori_loop(0, nchunk, body, 0)

    return gather_kernel, num_workers, nchunk


def kernel(x, table):
    batch, hist = x.shape
    vocab, embed_dim = table.shape
    batch_total = batch * hist
    gather_kernel, num_workers, nchunk = _make_gather(batch_total, vocab, embed_dim)
    idx = x.astype(jnp.int32).reshape(num_workers, nchunk, _CHUNK)
    out = gather_kernel(idx, table)
    return out.reshape(batch, hist, embed_dim)


# speedup vs baseline: 3.3426x; 1.1274x over previous
import functools

import jax
import jax.numpy as jnp
from jax import lax
from jax.experimental import pallas as pl
from jax.experimental.pallas import tpu as pltpu
from jax.experimental.pallas import tpu_sc as plsc

_EMBED_DIM = 128
_CHUNK = 128


@functools.lru_cache(maxsize=None)
def _make_gather(batch_total, vocab, embed_dim):
    info = plsc.get_sparse_core_info()
    num_cores = info.num_cores
    num_workers = info.num_cores * info.num_subcores
    b_per_w = batch_total // num_workers
    nchunk = b_per_w // _CHUNK
    assert b_per_w % _CHUNK == 0

    nbuf = 5
    ngroup = nchunk // nbuf
    assert nchunk % nbuf == 0

    mesh = plsc.VectorSubcoreMesh(core_axis_name="c", subcore_axis_name="s")

    @functools.partial(
        pl.kernel,
        mesh=mesh,
        out_type=jax.ShapeDtypeStruct((batch_total, embed_dim), jnp.float32),
        scratch_types=[
            pltpu.VMEM((nchunk, _CHUNK), jnp.int32),
            *[pltpu.VMEM((_CHUNK, embed_dim), jnp.float32) for _ in range(nbuf)],
            *[pltpu.SemaphoreType.DMA for _ in range(nbuf)],
        ],
    )
    def gather_kernel(idx_hbm, table_hbm, out_hbm, idx_v, *scratch):
        bufs = scratch[:nbuf]
        sems = scratch[nbuf:]
        wid = lax.axis_index("s") * num_cores + lax.axis_index("c")
        base = wid * b_per_w
        pltpu.sync_copy(idx_hbm.at[wid], idx_v)

        for b in range(nbuf):
            pltpu.async_copy(table_hbm.at[idx_v.at[b]], bufs[b], sems[b])

        def wait_gather(b):
            pltpu.make_async_copy(
                table_hbm.at[pl.ds(0, _CHUNK)], bufs[b], sems[b]
            ).wait()

        def body(j, carry):
            g0 = j * nbuf
            for b in range(nbuf):
                wait_gather(b)
                pltpu.sync_copy(
                    bufs[b], out_hbm.at[pl.ds(base + (g0 + b) * _CHUNK, _CHUNK)]
                )
                pltpu.async_copy(
                    table_hbm.at[idx_v.at[g0 + b + nbuf]], bufs[b], sems[b]
                )
            return carry

        lax.fori_loop(0, ngroup - 1, body, 0)

        g0 = (ngroup - 1) * nbuf
        for b in range(nbuf):
            wait_gather(b)
            pltpu.sync_copy(
                bufs[b], out_hbm.at[pl.ds(base + (g0 + b) * _CHUNK, _CHUNK)]
            )

    return gather_kernel, num_workers, nchunk


def kernel(x, table):
    batch, hist = x.shape
    vocab, embed_dim = table.shape
    batch_total = batch * hist
    gather_kernel, num_workers, nchunk = _make_gather(batch_total, vocab, embed_dim)
    idx = x.astype(jnp.int32).reshape(num_workers, nchunk, _CHUNK)
    out = gather_kernel(idx, table)
    return out.reshape(batch, hist, embed_dim)


# speedup vs baseline: 5.8251x; 1.7427x over previous
import functools

import jax
import jax.numpy as jnp
from jax import lax
from jax.experimental import pallas as pl
from jax.experimental.pallas import tpu as pltpu
from jax.experimental.pallas import tpu_sc as plsc

_GROUP_B = 8


@functools.lru_cache(maxsize=None)
def _make_gather(batch, hist, vocab, embed_dim):
    info = plsc.get_sparse_core_info()
    num_cores = info.num_cores
    num_workers = info.num_cores * info.num_subcores
    b_per_w = batch // num_workers
    rows_per_w = b_per_w * hist
    group_rows = _GROUP_B * hist
    ngroups = b_per_w // _GROUP_B
    assert batch % num_workers == 0 and b_per_w % _GROUP_B == 0

    chunk_offs = []
    off = 0
    while off < group_rows:
        n = min(128, group_rows - off)
        chunk_offs.append((off, n))
        off += n

    mesh = plsc.VectorSubcoreMesh(core_axis_name="c", subcore_axis_name="s")

    @functools.partial(
        pl.kernel,
        mesh=mesh,
        out_type=jax.ShapeDtypeStruct((batch, hist, embed_dim), jnp.float32),
        compiler_params=pltpu.CompilerParams(use_tc_tiling_on_sc=True),
        scratch_types=[
            pltpu.VMEM((rows_per_w,), jnp.int32),
            pltpu.VMEM((group_rows, embed_dim), jnp.float32),
            pltpu.VMEM((group_rows, embed_dim), jnp.float32),
            pltpu.SemaphoreType.DMA,
            pltpu.SemaphoreType.DMA,
            pltpu.SemaphoreType.DMA,
            pltpu.SemaphoreType.DMA,
        ],
    )
    def gather_kernel(idx_hbm, table_hbm, out_hbm, idx_v, buf_a, buf_b,
                      gsem_a, gsem_b, wsem_a, wsem_b):
        wid = lax.axis_index("s") * num_cores + lax.axis_index("c")
        base_b = wid * b_per_w
        pltpu.sync_copy(idx_hbm.at[pl.ds(wid * rows_per_w, rows_per_w)], idx_v)

        def start_gather(g, buf, gsem):
            gbase = g * group_rows
            for off, n in chunk_offs:
                pltpu.async_copy(
                    table_hbm.at[idx_v.at[pl.ds(gbase + off, n)]],
                    buf.at[pl.ds(off, n)], gsem)

        def wait_gather(buf, gsem):
            for off, n in chunk_offs:
                pltpu.make_async_copy(
                    table_hbm.at[pl.ds(0, n)], buf.at[pl.ds(off, n)], gsem
                ).wait()

        def start_writes(g, buf, wsem):
            for j in range(_GROUP_B):
                pltpu.async_copy(
                    buf.at[pl.ds(j * hist, hist)],
                    out_hbm.at[base_b + g * _GROUP_B + j], wsem)

        def wait_writes(buf, wsem):
            for j in range(_GROUP_B):
                pltpu.make_async_copy(
                    buf.at[pl.ds(j * hist, hist)], out_hbm.at[0], wsem
                ).wait()

        start_gather(0, buf_a, gsem_a)
        wait_gather(buf_a, gsem_a)
        start_writes(0, buf_a, wsem_a)
        start_gather(1, buf_b, gsem_b)

        def pair(j, carry):
            g1 = 2 * j + 1
            wait_gather(buf_b, gsem_b)
            start_writes(g1, buf_b, wsem_b)
            wait_writes(buf_a, wsem_a)
            start_gather(g1 + 1, buf_a, gsem_a)
            wait_gather(buf_a, gsem_a)
            start_writes(g1 + 1, buf_a, wsem_a)
            wait_writes(buf_b, wsem_b)
            start_gather(g1 + 2, buf_b, gsem_b)
            return carry

        lax.fori_loop(0, (ngroups - 2) // 2, pair, 0)

        wait_gather(buf_b, gsem_b)
        start_writes(ngroups - 1, buf_b, wsem_b)
        wait_writes(buf_a, wsem_a)
        wait_writes(buf_b, wsem_b)

    return gather_kernel, num_workers


def kernel(x, table):
    batch, hist = x.shape
    vocab, embed_dim = table.shape
    gather_kernel, _ = _make_gather(batch, hist, vocab, embed_dim)
    idx = x.astype(jnp.int32).reshape(-1)
    return gather_kernel(idx, table)


# speedup vs baseline: 10.2841x; 1.7655x over previous
import functools

import jax
import jax.numpy as jnp
from jax import lax
from jax.experimental import pallas as pl
from jax.experimental.pallas import tpu as pltpu
from jax.experimental.pallas import tpu_sc as plsc

_CHUNK = 64
_GROUP = 5


@functools.lru_cache(maxsize=None)
def _make_gather(batch, hist, vocab, embed_dim):
    info = plsc.get_sparse_core_info()
    num_cores = info.num_cores
    num_workers = info.num_cores * info.num_subcores
    batch_total = batch * hist
    b_per_w = batch // num_workers
    rows_per_w = b_per_w * hist
    nchunk = rows_per_w // _CHUNK
    chunks_per_h = b_per_w // _CHUNK
    assert batch % num_workers == 0 and b_per_w % _CHUNK == 0
    ngroups = nchunk // _GROUP
    assert nchunk % _GROUP == 0

    mesh = plsc.VectorSubcoreMesh(core_axis_name="c", subcore_axis_name="s")

    @functools.partial(
        pl.kernel,
        mesh=mesh,
        out_type=jax.ShapeDtypeStruct((batch_total, embed_dim), jnp.float32),
        compiler_params=pltpu.CompilerParams(use_tc_tiling_on_sc=True),
        scratch_types=[
            pltpu.VMEM((rows_per_w,), jnp.int32),
            pltpu.VMEM((_GROUP * _CHUNK, embed_dim), jnp.float32),
            pltpu.VMEM((_GROUP * _CHUNK, embed_dim), jnp.float32),
            pltpu.SemaphoreType.DMA,
            pltpu.SemaphoreType.DMA,
            pltpu.SemaphoreType.DMA,
            pltpu.SemaphoreType.DMA,
        ],
    )
    def gather_kernel(idx_hbm, table_hbm, out_hbm, idx_v, buf_a, buf_b,
                      gsem_a, gsem_b, wsem_a, wsem_b):
        wid = lax.axis_index("s") * num_cores + lax.axis_index("c")
        pltpu.sync_copy(idx_hbm.at[pl.ds(wid * rows_per_w, rows_per_w)], idx_v)

        def start_gather(g, buf, gsem):
            for k in range(_GROUP):
                pltpu.async_copy(
                    table_hbm.at[
                        idx_v.at[pl.ds((g * _GROUP + k) * _CHUNK, _CHUNK)]],
                    buf.at[pl.ds(k * _CHUNK, _CHUNK)], gsem)

        def wait_gather(buf, gsem):
            for k in range(_GROUP):
                pltpu.make_async_copy(
                    table_hbm.at[pl.ds(0, _CHUNK)],
                    buf.at[pl.ds(k * _CHUNK, _CHUNK)], gsem).wait()

        def start_write(g, buf, wsem):
            for k in range(_GROUP):
                c = g * _GROUP + k
                row = ((c // chunks_per_h) * batch + wid * b_per_w
                       + (c % chunks_per_h) * _CHUNK)
                pltpu.async_copy(
                    buf.at[pl.ds(k * _CHUNK, _CHUNK)],
                    out_hbm.at[pl.ds(row, _CHUNK)], wsem)

        def wait_write(buf, wsem):
            for k in range(_GROUP):
                pltpu.make_async_copy(
                    buf.at[pl.ds(k * _CHUNK, _CHUNK)],
                    out_hbm.at[pl.ds(0, _CHUNK)], wsem).wait()

        start_gather(0, buf_a, gsem_a)
        wait_gather(buf_a, gsem_a)
        start_write(0, buf_a, wsem_a)
        start_gather(1, buf_b, gsem_b)

        def pair(j, carry):
            g1 = 2 * j + 1
            wait_gather(buf_b, gsem_b)
            start_write(g1, buf_b, wsem_b)
            wait_write(buf_a, wsem_a)
            start_gather(g1 + 1, buf_a, gsem_a)
            wait_gather(buf_a, gsem_a)
            start_write(g1 + 1, buf_a, wsem_a)
            wait_write(buf_b, wsem_b)
            start_gather(g1 + 2, buf_b, gsem_b)
            return carry

        lax.fori_loop(0, (ngroups - 2) // 2, pair, 0)

        if ngroups % 2 == 1:
            wait_gather(buf_b, gsem_b)
            start_write(ngroups - 2, buf_b, wsem_b)
            wait_write(buf_a, wsem_a)
            start_gather(ngroups - 1, buf_a, gsem_a)
            wait_gather(buf_a, gsem_a)
            start_write(ngroups - 1, buf_a, wsem_a)
            wait_write(buf_b, wsem_b)
            wait_write(buf_a, wsem_a)
        else:
            wait_gather(buf_b, gsem_b)
            start_write(ngroups - 1, buf_b, wsem_b)
            wait_write(buf_a, wsem_a)
            wait_write(buf_b, wsem_b)

    return gather_kernel, num_workers, b_per_w


def kernel(x, table):
    batch, hist = x.shape
    vocab, embed_dim = table.shape
    gather_kernel, num_workers, b_per_w = _make_gather(batch, hist, vocab, embed_dim)
    idx = (
        x.astype(jnp.int32)
        .reshape(num_workers, b_per_w, hist)
        .transpose(0, 2, 1)
        .reshape(-1)
    )
    out = gather_kernel(idx, table)
    return out.reshape(hist, batch, embed_dim).transpose(1, 0, 2)
